# baseline (speedup 1.0000x reference)
import functools

import jax
import jax.numpy as jnp
from jax import lax
from jax.experimental import pallas as pl
from jax.experimental.pallas import tpu as pltpu
from jax.experimental.pallas import tpu_sc as plsc

_B, _T, _D = 4096, 200, 64
_L = 16
_NDB = _D // 8
_NQ = 32 // _NDB
_TPQ = _T // _NQ
_G = _B // _L


def _compute_slab(t, dblk, te_v, bt_v, obuf):
    quads = [te_v[t, pl.ds(k * _L, _L)] for k in range(4)]
    j = dblk // 2
    tvec = jnp.where(j == 0, quads[0],
                     jnp.where(j == 1, quads[1],
                               jnp.where(j == 2, quads[2], quads[3])))
    odd = (dblk % 2) == 1
    tv = [jnp.full((_L,), jnp.where(odd, tvec[din + 8], tvec[din]))
          for din in range(8)]

    @plsc.parallel_loop(0, _G, unroll=4)
    def _(g):
        s = g * _L
        for din in range(8):
            obuf[din, pl.ds(s, _L)] = bt_v[din, pl.ds(s, _L)] + tv[din]


def _sc_body(bt_hbm, te_hbm, out_hbm, bt_v, te_v, obuf0, obuf1, sem0, sem1):
    nc = 2
    wid = lax.axis_index("s") * nc + lax.axis_index("c")
    dblk = wid % _NDB
    t0 = (wid // _NDB) * _TPQ
    pltpu.sync_copy(bt_hbm.at[pl.ds(dblk * 8, 8), :], bt_v)
    pltpu.sync_copy(te_hbm, te_v)

    def pair(i, c):
        ta = t0 + 2 * i
        tb = ta + 1

        @pl.when(i > 0)
        def _():
            pltpu.make_async_copy(
                obuf0, out_hbm.at[t0, pl.ds(dblk * 8, 8), :], sem0).wait()

        _compute_slab(ta, dblk, te_v, bt_v, obuf0)
        pltpu.async_copy(obuf0, out_hbm.at[ta, pl.ds(dblk * 8, 8), :], sem0)

        @pl.when(i > 0)
        def _():
            pltpu.make_async_copy(
                obuf1, out_hbm.at[t0, pl.ds(dblk * 8, 8), :], sem1).wait()

        _compute_slab(tb, dblk, te_v, bt_v, obuf1)
        pltpu.async_copy(obuf1, out_hbm.at[tb, pl.ds(dblk * 8, 8), :], sem1)
        return c

    lax.fori_loop(0, _TPQ // 2, pair, 0)
    pltpu.make_async_copy(obuf0, out_hbm.at[0, pl.ds(0, 8), :], sem0).wait()
    pltpu.make_async_copy(obuf1, out_hbm.at[0, pl.ds(0, 8), :], sem1).wait()


_sc_kernel = functools.partial(
    pl.kernel,
    out_type=jax.ShapeDtypeStruct((_T, _D, _B), jnp.float32),
    mesh=plsc.VectorSubcoreMesh(core_axis_name="c", subcore_axis_name="s"),
    scratch_types=[
        pltpu.VMEM((8, _B), jnp.float32),
        pltpu.VMEM((_T, _D), jnp.float32),
        pltpu.VMEM((8, _B), jnp.float32),
        pltpu.VMEM((8, _B), jnp.float32),
        pltpu.SemaphoreType.DMA,
        pltpu.SemaphoreType.DMA,
    ],
    compiler_params=pltpu.CompilerParams(use_tc_tiling_on_sc=True),
)(_sc_body)


def kernel(x, batch_embed, token_embed):
    del x
    bt = batch_embed.T
    z = _sc_kernel(bt, token_embed)
    return jnp.transpose(z, (2, 0, 1))


# speedup vs baseline: 1.0227x; 1.0227x over previous
import functools

import jax
import jax.numpy as jnp
from jax import lax
from jax.experimental import pallas as pl
from jax.experimental.pallas import tpu as pltpu
from jax.experimental.pallas import tpu_sc as plsc

_B, _T, _D = 4096, 200, 64
_L = 16
_NDB = _D // 8
_NQ = 32 // _NDB
_TPQ = _T // _NQ
_G = _B // _L


def _compute_slab(t, tep_v, bt_v, obuf):
    tvec = tep_v[t, :]
    tv = [jnp.full((_L,), tvec[din]) for din in range(8)]

    @plsc.parallel_loop(0, _G, unroll=8)
    def _(g):
        s = g * _L
        for din in range(8):
            obuf[din, pl.ds(s, _L)] = bt_v[din, pl.ds(s, _L)] + tv[din]


def _sc_body(bt_hbm, tep_hbm, out_hbm, bt_v, tep_v, obuf0, obuf1, sem0, sem1):
    nc = 2
    wid = lax.axis_index("s") * nc + lax.axis_index("c")
    dblk = wid % _NDB
    t0 = (wid // _NDB) * _TPQ
    pltpu.sync_copy(bt_hbm.at[pl.ds(dblk * 8, 8), :], bt_v)
    pltpu.sync_copy(tep_hbm.at[dblk], tep_v)

    def pair(i, c):
        ta = t0 + 2 * i
        tb = ta + 1

        @pl.when(i > 0)
        def _():
            pltpu.make_async_copy(
                obuf0, out_hbm.at[t0, pl.ds(dblk * 8, 8), :], sem0).wait()

        _compute_slab(ta, tep_v, bt_v, obuf0)
        pltpu.async_copy(obuf0, out_hbm.at[ta, pl.ds(dblk * 8, 8), :], sem0)

        @pl.when(i > 0)
        def _():
            pltpu.make_async_copy(
                obuf1, out_hbm.at[t0, pl.ds(dblk * 8, 8), :], sem1).wait()

        _compute_slab(tb, tep_v, bt_v, obuf1)
        pltpu.async_copy(obuf1, out_hbm.at[tb, pl.ds(dblk * 8, 8), :], sem1)
        return c

    lax.fori_loop(0, _TPQ // 2, pair, 0)
    pltpu.make_async_copy(obuf0, out_hbm.at[0, pl.ds(0, 8), :], sem0).wait()
    pltpu.make_async_copy(obuf1, out_hbm.at[0, pl.ds(0, 8), :], sem1).wait()


_sc_kernel = functools.partial(
    pl.kernel,
    out_type=jax.ShapeDtypeStruct((_T, _D, _B), jnp.float32),
    mesh=plsc.VectorSubcoreMesh(core_axis_name="c", subcore_axis_name="s"),
    scratch_types=[
        pltpu.VMEM((8, _B), jnp.float32),
        pltpu.VMEM((_T, _L), jnp.float32),
        pltpu.VMEM((8, _B), jnp.float32),
        pltpu.VMEM((8, _B), jnp.float32),
        pltpu.SemaphoreType.DMA,
        pltpu.SemaphoreType.DMA,
    ],
    compiler_params=pltpu.CompilerParams(use_tc_tiling_on_sc=True),
)(_sc_body)


def kernel(x, batch_embed, token_embed):
    del x
    bt = batch_embed.T
    tep = token_embed.reshape(_T, _NDB, 8).transpose(1, 0, 2)
    tep = jnp.pad(tep, ((0, 0), (0, 0), (0, _L - 8)))
    z = _sc_kernel(bt, tep)
    return jnp.transpose(z, (2, 0, 1))
